# Initial kernel scaffold; baseline (speedup 1.0000x reference)
import jax
import jax.numpy as jnp
from jax.experimental import pallas as pl


def kernel(n_id, edge_index, edge_types, emb_table, bn32_g, bn32_b, W1, root1, b1, bn64_g, bn64_b, W2, root2, b2, bn64_2_g, bn64_2_b):
    raise NotImplementedError("write your pallas kernel here")


# speedup vs baseline: 3.0087x; 3.0087x over previous
import functools

import jax
import jax.numpy as jnp
from jax import lax
from jax.experimental import pallas as pl
from jax.experimental.pallas import tpu as pltpu
from jax.experimental.pallas import tpu_sc as plsc

NC, NS, L = 2, 16, 16
NW = NC * NS

N = 50000
E = 800000
R = 16
D_IN = 32
D_HID = 64
EPS = 1e-5

NPAD = 50176
EPAD = 800768
EC = 1472

HLOC = 400128
HREAL = N * R // 2
HDUMMY = 400064
BN = 512

_mesh = plsc.VectorSubcoreMesh(
    core_axis_name="c", subcore_axis_name="s", num_cores=NC, num_subcores=NS)
_sc_params = pltpu.CompilerParams(use_tc_tiling_on_sc=False)


def _emb_gather(table, idx):
    b_per_w = NPAD // NW

    @functools.partial(
        pl.kernel, mesh=_mesh,
        out_type=jax.ShapeDtypeStruct((NPAD, D_IN), jnp.float32),
        compiler_params=_sc_params,
        scratch_types=[
            pltpu.VMEM((b_per_w,), jnp.int32),
            pltpu.VMEM((b_per_w, D_IN), jnp.float32),
            pltpu.SemaphoreType.DMA,
        ],
    )
    def k(table_hbm, idx_hbm, out_hbm, idx_v, rows_v, sem):
        wid = lax.axis_index("s") * NC + lax.axis_index("c")
        base = wid * b_per_w
        pltpu.sync_copy(idx_hbm.at[pl.ds(base, b_per_w)], idx_v)
        pltpu.async_copy(table_hbm.at[idx_v], rows_v, sem).wait()
        pltpu.sync_copy(rows_v, out_hbm.at[pl.ds(base, b_per_w)])

    return k(table, idx)


def _seg_counts(dstp, etp):
    @functools.partial(
        pl.kernel, mesh=_mesh,
        out_type=jax.ShapeDtypeStruct((N * R + 8,), jnp.float32),
        compiler_params=_sc_params,
        scratch_types=[
            pltpu.VMEM_SHARED((HLOC,), jnp.float32),
            pltpu.VMEM((HLOC // NS,), jnp.float32),
            pltpu.VMEM((EC,), jnp.int32),
            pltpu.VMEM((EC,), jnp.int32),
            pltpu.VMEM((EC,), jnp.int32),
            pltpu.VMEM((EC,), jnp.float32),
        ],
    )
    def k(dst_hbm, et_hbm, cnt_hbm, hist, zbuf, dstv, etv, idxv, onesv):
        c = lax.axis_index("c")
        tec = lax.axis_index("s")
        zslice = HLOC // NS

        def z16(i, _):
            zbuf[pl.ds(i * 16, 16)] = jnp.zeros((16,), jnp.float32)
            return 0
        lax.fori_loop(0, zslice // 16, z16, 0)
        pltpu.sync_copy(zbuf, hist.at[pl.ds(tec * zslice, zslice)])

        def o16(i, _):
            onesv[pl.ds(i * 16, 16)] = jnp.ones((16,), jnp.float32)
            return 0
        lax.fori_loop(0, EC // 16, o16, 0)
        plsc.subcore_barrier()

        lo = c * HREAL
        ebase = tec * (EPAD // NS)

        def chunk(ch, _):
            cb = ebase + ch * EC
            pltpu.sync_copy(dst_hbm.at[pl.ds(cb, EC)], dstv)
            pltpu.sync_copy(et_hbm.at[pl.ds(cb, EC)], etv)

            def cmp16(i, _):
                sl = pl.ds(i * 16, 16)
                seg = dstv[sl] * R + etv[sl] - lo
                ok = (seg >= 0) & (seg < HREAL)
                idxv[sl] = jnp.where(ok, seg, HDUMMY)
                return 0
            lax.fori_loop(0, EC // 16, cmp16, 0)
            pltpu.sync_copy(onesv, hist.at[idxv], add=True)
            return 0
        lax.fori_loop(0, EPAD // NS // EC, chunk, 0)
        plsc.subcore_barrier()

        wslice = HREAL // NS
        pltpu.sync_copy(hist.at[pl.ds(tec * wslice, wslice)],
                        cnt_hbm.at[pl.ds(c * HREAL + tec * wslice, wslice)])

    return k(dstp, etp)


def _edge_weights(srcp, dstp, etp, cnt):
    e_per_w = EPAD // NW

    @functools.partial(
        pl.kernel, mesh=_mesh,
        out_type=[jax.ShapeDtypeStruct((EPAD,), jnp.float32),
                  jax.ShapeDtypeStruct((EPAD,), jnp.int32)],
        compiler_params=_sc_params,
        scratch_types=[
            pltpu.VMEM((EC,), jnp.int32),
            pltpu.VMEM((EC,), jnp.int32),
            pltpu.VMEM((EC,), jnp.int32),
            pltpu.VMEM((EC,), jnp.int32),
            pltpu.VMEM((EC,), jnp.float32),
            pltpu.VMEM((EC,), jnp.float32),
            pltpu.VMEM((EC,), jnp.int32),
            pltpu.SemaphoreType.DMA,
        ],
    )
    def k(src_hbm, dst_hbm, et_hbm, cnt_hbm, w_hbm, g_hbm,
          srcv, dstv, etv, segv, cv, wv, gvb, sem):
        wid = lax.axis_index("s") * NC + lax.axis_index("c")
        wbase = wid * e_per_w

        def chunk(ch, _):
            cb = wbase + ch * EC
            pltpu.sync_copy(src_hbm.at[pl.ds(cb, EC)], srcv)
            pltpu.sync_copy(dst_hbm.at[pl.ds(cb, EC)], dstv)
            pltpu.sync_copy(et_hbm.at[pl.ds(cb, EC)], etv)

            def cmp16(i, _):
                sl = pl.ds(i * 16, 16)
                segv[sl] = dstv[sl] * R + etv[sl]
                gvb[sl] = etv[sl] * NPAD + srcv[sl]
                return 0
            lax.fori_loop(0, EC // 16, cmp16, 0)
            pltpu.async_copy(cnt_hbm.at[segv], cv, sem).wait()

            def inv16(i, _):
                sl = pl.ds(i * 16, 16)
                wv[sl] = 1.0 / cv[sl]
                return 0
            lax.fori_loop(0, EC // 16, inv16, 0)
            pltpu.sync_copy(wv, w_hbm.at[pl.ds(cb, EC)])
            pltpu.sync_copy(gvb, g_hbm.at[pl.ds(cb, EC)])
            return 0
        lax.fori_loop(0, e_per_w // EC, chunk, 0)

    return k(srcp, dstp, etp, cnt)


def _tc_transform(x, Wcat, bias, spre, tpre, relu_pre):
    d_in = x.shape[1]

    def body(x_ref, w_ref, b_ref, s_ref, t_ref, y_ref, z_ref):
        xb = x_ref[...]
        if relu_pre:
            xb = jnp.maximum(xb, 0.0)
        xb = xb * s_ref[...] + t_ref[...]
        ycat = jnp.dot(xb, w_ref[...], preferred_element_type=jnp.float32)
        for h in range(2):
            for r in range(R):
                y_ref[h, r] = ycat[:, 512 * h + 32 * r: 512 * h + 32 * r + 32]
            z_ref[h] = (ycat[:, 1024 + 32 * h: 1056 + 32 * h]
                        + b_ref[0, 32 * h: 32 * h + 32])

    return pl.pallas_call(
        body,
        grid=(NPAD // BN,),
        in_specs=[pl.BlockSpec((BN, d_in), lambda i: (i, 0)),
                  pl.BlockSpec((d_in, 1088), lambda i: (0, 0)),
                  pl.BlockSpec((1, D_HID), lambda i: (0, 0)),
                  pl.BlockSpec((1, d_in), lambda i: (0, 0)),
                  pl.BlockSpec((1, d_in), lambda i: (0, 0))],
        out_specs=[pl.BlockSpec((2, R, BN, 32), lambda i: (0, 0, i, 0)),
                   pl.BlockSpec((2, BN, 32), lambda i: (0, i, 0))],
        out_shape=[jax.ShapeDtypeStruct((2, R, NPAD, 32), jnp.float32),
                   jax.ShapeDtypeStruct((2, NPAD, 32), jnp.float32)],
    )(x, Wcat, bias.reshape(1, D_HID), spre.reshape(1, d_in),
      tpre.reshape(1, d_in))


def _sc_aggregate(Yflat, Z, dstp, gp, wp, sfin, tfin):
    rows_t = NPAD // NS
    drows = 224
    eca = 544

    @functools.partial(
        pl.kernel, mesh=_mesh,
        out_type=jax.ShapeDtypeStruct((NPAD, D_HID), jnp.float32),
        compiler_params=_sc_params,
        scratch_types=[
            pltpu.VMEM_SHARED((NPAD, 32), jnp.float32),
            pltpu.VMEM((eca,), jnp.int32),
            pltpu.VMEM((eca,), jnp.int32),
            pltpu.VMEM((eca + 16,), jnp.float32),
            pltpu.VMEM((eca, 32), jnp.float32),
            pltpu.VMEM((drows, 32), jnp.float32),
            pltpu.VMEM((32,), jnp.float32),
            pltpu.VMEM((32,), jnp.float32),
            pltpu.SemaphoreType.DMA,
        ],
    )
    def k(y_hbm, z_hbm, dst_hbm, g_hbm, w_hbm, s_hbm, t_hbm, out_hbm,
          acc, dstv, gv, wv, msg, dv, svv, tvv, sem):
        h = lax.axis_index("c")
        tec = lax.axis_index("s")
        rowb = tec * rows_t
        pltpu.sync_copy(z_hbm.at[h, pl.ds(rowb, rows_t)],
                        acc.at[pl.ds(rowb, rows_t)])
        pltpu.sync_copy(s_hbm.at[h], svv)
        pltpu.sync_copy(t_hbm.at[h], tvv)
        plsc.subcore_barrier()

        ebase = tec * (EPAD // NS)
        yoff = h * (R * NPAD)

        def chunk(ch, _):
            cb = ebase + ch * eca
            pltpu.sync_copy(dst_hbm.at[pl.ds(cb, eca)], dstv)
            pltpu.sync_copy(g_hbm.at[pl.ds(cb, eca)], gv)
            pltpu.sync_copy(w_hbm.at[pl.ds(cb, eca)], wv.at[pl.ds(0, eca)])

            def addoff(i, _):
                sl = pl.ds(i * 16, 16)
                gv[sl] = gv[sl] + yoff
                return 0
            lax.fori_loop(0, eca // 16, addoff, 0)
            pltpu.async_copy(y_hbm.at[gv], msg, sem).wait()

            def scale(i, _):
                wsc = wv[pl.ds(i, 16)][0]
                msg[i, 0:16] = msg[i, 0:16] * wsc
                msg[i, 16:32] = msg[i, 16:32] * wsc
                return 0
            lax.fori_loop(0, eca, scale, 0)
            pltpu.sync_copy(msg, acc.at[dstv], add=True)
            return 0
        lax.fori_loop(0, EPAD // NS // eca, chunk, 0)
        plsc.subcore_barrier()

        slo, shi = svv[0:16], svv[16:32]
        tlo, thi = tvv[0:16], tvv[16:32]

        def drain(d, _):
            rb = rowb + d * drows
            pltpu.sync_copy(acc.at[pl.ds(rb, drows)], dv)

            def bnrow(i, _):
                dv[i, 0:16] = dv[i, 0:16] * slo + tlo
                dv[i, 16:32] = dv[i, 16:32] * shi + thi
                return 0
            lax.fori_loop(0, drows, bnrow, 0)
            pltpu.sync_copy(dv, out_hbm.at[pl.ds(rb, drows),
                                           pl.ds(h * 32, 32)])
            return 0
        lax.fori_loop(0, rows_t // drows, drain, 0)

    return k(Yflat, Z, dstp, gp, wp, sfin, tfin)


def _pack_weights(W, root):
    halves = []
    for h in range(2):
        halves.append(jnp.concatenate(
            [W[r][:, h * 32:(h + 1) * 32] for r in range(R)], axis=1))
    return jnp.concatenate(halves + [root], axis=1)


def kernel(n_id, edge_index, edge_types, emb_table,
           bn32_g, bn32_b, W1, root1, b1, bn64_g, bn64_b,
           W2, root2, b2, bn64_2_g, bn64_2_b):
    inv = 1.0 / jnp.sqrt(1.0 + EPS)
    idx = jnp.pad(n_id.astype(jnp.int32), (0, NPAD - N))
    srcp = jnp.pad(edge_index[0].astype(jnp.int32), (0, EPAD - E))
    dstp = jnp.pad(edge_index[1].astype(jnp.int32), (0, EPAD - E),
                   constant_values=N)
    etp = jnp.pad(edge_types[:E].astype(jnp.int32), (0, EPAD - E))

    x0 = _emb_gather(emb_table, idx)
    cnt = _seg_counts(dstp, etp)
    w, g = _edge_weights(srcp, dstp, etp, cnt)

    ones2 = jnp.ones((2, 32), jnp.float32)
    zeros2 = jnp.zeros((2, 32), jnp.float32)
    sfin = (bn64_2_g * inv).reshape(2, 32)
    tfin = bn64_2_b.reshape(2, 32)

    Y1, Z1 = _tc_transform(x0, _pack_weights(W1, root1), b1,
                           bn32_g * inv, bn32_b, relu_pre=False)
    h1 = _sc_aggregate(Y1.reshape(2 * R * NPAD, 32), Z1,
                       dstp, g, w, ones2, zeros2)

    Y2, Z2 = _tc_transform(h1, _pack_weights(W2, root2), b2,
                           bn64_g * inv, bn64_b, relu_pre=True)
    h2 = _sc_aggregate(Y2.reshape(2 * R * NPAD, 32), Z2,
                       dstp, g, w, sfin, tfin)
    return h2[:N]


# speedup vs baseline: 5.2945x; 1.7597x over previous
import functools

import jax
import jax.numpy as jnp
from jax import lax
from jax.experimental import pallas as pl
from jax.experimental.pallas import tpu as pltpu
from jax.experimental.pallas import tpu_sc as plsc

NC, NS, L = 2, 16, 16
NW = NC * NS

N = 50000
E = 800000
R = 16
D_IN = 32
D_HID = 64
EPS = 1e-5

NPAD = 50176
EPAD = 800768
EC = 1472

HLOC = 400128
HREAL = N * R // 2
HDUMMY = 400064
BN = 512

_mesh = plsc.VectorSubcoreMesh(
    core_axis_name="c", subcore_axis_name="s", num_cores=NC, num_subcores=NS)
_sc_params = pltpu.CompilerParams(use_tc_tiling_on_sc=False)


def _emb_gather(table, idx):
    b_per_w = NPAD // NW

    @functools.partial(
        pl.kernel, mesh=_mesh,
        out_type=jax.ShapeDtypeStruct((NPAD, D_IN), jnp.float32),
        compiler_params=_sc_params,
        scratch_types=[
            pltpu.VMEM((b_per_w,), jnp.int32),
            pltpu.VMEM((b_per_w, D_IN), jnp.float32),
            pltpu.SemaphoreType.DMA,
        ],
    )
    def k(table_hbm, idx_hbm, out_hbm, idx_v, rows_v, sem):
        wid = lax.axis_index("s") * NC + lax.axis_index("c")
        base = wid * b_per_w
        pltpu.sync_copy(idx_hbm.at[pl.ds(base, b_per_w)], idx_v)
        pltpu.async_copy(table_hbm.at[idx_v], rows_v, sem).wait()
        pltpu.sync_copy(rows_v, out_hbm.at[pl.ds(base, b_per_w)])

    return k(table, idx)


def _seg_counts(dstp, etp):
    @functools.partial(
        pl.kernel, mesh=_mesh,
        out_type=jax.ShapeDtypeStruct((N * R + 8,), jnp.float32),
        compiler_params=_sc_params,
        scratch_types=[
            pltpu.VMEM_SHARED((HLOC,), jnp.float32),
            pltpu.VMEM((HLOC // NS,), jnp.float32),
            pltpu.VMEM((EC,), jnp.int32),
            pltpu.VMEM((EC,), jnp.int32),
            pltpu.VMEM((EC,), jnp.int32),
            pltpu.VMEM((EC,), jnp.float32),
        ],
    )
    def k(dst_hbm, et_hbm, cnt_hbm, hist, zbuf, dstv, etv, idxv, onesv):
        c = lax.axis_index("c")
        tec = lax.axis_index("s")
        zslice = HLOC // NS

        def z16(i, _):
            zbuf[pl.ds(i * 16, 16)] = jnp.zeros((16,), jnp.float32)
            return 0
        lax.fori_loop(0, zslice // 16, z16, 0)
        pltpu.sync_copy(zbuf, hist.at[pl.ds(tec * zslice, zslice)])

        def o16(i, _):
            onesv[pl.ds(i * 16, 16)] = jnp.ones((16,), jnp.float32)
            return 0
        lax.fori_loop(0, EC // 16, o16, 0)
        plsc.subcore_barrier()

        lo = c * HREAL
        ebase = tec * (EPAD // NS)

        def chunk(ch, _):
            cb = ebase + ch * EC
            pltpu.sync_copy(dst_hbm.at[pl.ds(cb, EC)], dstv)
            pltpu.sync_copy(et_hbm.at[pl.ds(cb, EC)], etv)

            def cmp16(i, _):
                sl = pl.ds(i * 16, 16)
                seg = dstv[sl] * R + etv[sl] - lo
                ok = (seg >= 0) & (seg < HREAL)
                idxv[sl] = jnp.where(ok, seg, HDUMMY)
                return 0
            lax.fori_loop(0, EC // 16, cmp16, 0)
            pltpu.sync_copy(onesv, hist.at[idxv], add=True)
            return 0
        lax.fori_loop(0, EPAD // NS // EC, chunk, 0)
        plsc.subcore_barrier()

        wslice = HREAL // NS
        pltpu.sync_copy(hist.at[pl.ds(tec * wslice, wslice)],
                        cnt_hbm.at[pl.ds(c * HREAL + tec * wslice, wslice)])

    return k(dstp, etp)


def _edge_weights(srcp, dstp, etp, cnt):
    e_per_w = EPAD // NW

    @functools.partial(
        pl.kernel, mesh=_mesh,
        out_type=[jax.ShapeDtypeStruct((EPAD,), jnp.float32),
                  jax.ShapeDtypeStruct((EPAD,), jnp.int32)],
        compiler_params=_sc_params,
        scratch_types=[
            pltpu.VMEM((EC,), jnp.int32),
            pltpu.VMEM((EC,), jnp.int32),
            pltpu.VMEM((EC,), jnp.int32),
            pltpu.VMEM((EC,), jnp.int32),
            pltpu.VMEM((EC,), jnp.float32),
            pltpu.VMEM((EC,), jnp.float32),
            pltpu.VMEM((EC,), jnp.int32),
            pltpu.SemaphoreType.DMA,
        ],
    )
    def k(src_hbm, dst_hbm, et_hbm, cnt_hbm, w_hbm, g_hbm,
          srcv, dstv, etv, segv, cv, wv, gvb, sem):
        wid = lax.axis_index("s") * NC + lax.axis_index("c")
        wbase = wid * e_per_w

        def chunk(ch, _):
            cb = wbase + ch * EC
            pltpu.sync_copy(src_hbm.at[pl.ds(cb, EC)], srcv)
            pltpu.sync_copy(dst_hbm.at[pl.ds(cb, EC)], dstv)
            pltpu.sync_copy(et_hbm.at[pl.ds(cb, EC)], etv)

            def cmp16(i, _):
                sl = pl.ds(i * 16, 16)
                segv[sl] = dstv[sl] * R + etv[sl]
                gvb[sl] = srcv[sl] * 32 + etv[sl]
                return 0
            lax.fori_loop(0, EC // 16, cmp16, 0)
            pltpu.async_copy(cnt_hbm.at[segv], cv, sem).wait()

            def inv16(i, _):
                sl = pl.ds(i * 16, 16)
                wv[sl] = 1.0 / cv[sl]
                return 0
            lax.fori_loop(0, EC // 16, inv16, 0)
            pltpu.sync_copy(wv, w_hbm.at[pl.ds(cb, EC)])
            pltpu.sync_copy(gvb, g_hbm.at[pl.ds(cb, EC)])
            return 0
        lax.fori_loop(0, e_per_w // EC, chunk, 0)

    return k(srcp, dstp, etp, cnt)


def _tc_transform(x, Wcat, bias, spre, tpre, relu_pre):
    d_in = x.shape[1]

    def body(x_ref, w_ref, b_ref, s_ref, t_ref, y_ref, z_ref):
        xb = x_ref[...]
        if relu_pre:
            xb = jnp.maximum(xb, 0.0)
        xb = xb * s_ref[...] + t_ref[...]
        ycat = jnp.dot(xb, w_ref[...], preferred_element_type=jnp.float32)
        y_ref[...] = ycat[:, 0:1024]
        for h in range(2):
            z_ref[h] = (ycat[:, 1024 + 32 * h: 1056 + 32 * h]
                        + b_ref[0, 32 * h: 32 * h + 32])

    return pl.pallas_call(
        body,
        grid=(NPAD // BN,),
        in_specs=[pl.BlockSpec((BN, d_in), lambda i: (i, 0)),
                  pl.BlockSpec((d_in, 1088), lambda i: (0, 0)),
                  pl.BlockSpec((1, D_HID), lambda i: (0, 0)),
                  pl.BlockSpec((1, d_in), lambda i: (0, 0)),
                  pl.BlockSpec((1, d_in), lambda i: (0, 0))],
        out_specs=[pl.BlockSpec((BN, 1024), lambda i: (i, 0)),
                   pl.BlockSpec((2, BN, 32), lambda i: (0, i, 0))],
        out_shape=[jax.ShapeDtypeStruct((NPAD, 1024), jnp.float32),
                   jax.ShapeDtypeStruct((2, NPAD, 32), jnp.float32)],
    )(x, Wcat, bias.reshape(1, D_HID), spre.reshape(1, d_in),
      tpre.reshape(1, d_in))


def _sc_aggregate(Yflat, Z, dstp, gp, wp, sfin, tfin):
    rows_t = NPAD // NS
    drows = 224
    eca = 544

    @functools.partial(
        pl.kernel, mesh=_mesh,
        out_type=jax.ShapeDtypeStruct((NPAD, D_HID), jnp.float32),
        compiler_params=_sc_params,
        scratch_types=[
            pltpu.VMEM_SHARED((NPAD, 32), jnp.float32),
            pltpu.VMEM((eca,), jnp.int32),
            pltpu.VMEM((eca,), jnp.int32),
            pltpu.VMEM((eca,), jnp.float32),
            pltpu.VMEM((eca, 32), jnp.float32),
            pltpu.VMEM((drows, 32), jnp.float32),
            pltpu.VMEM((32,), jnp.float32),
            pltpu.VMEM((32,), jnp.float32),
            pltpu.SemaphoreType.DMA,
        ],
    )
    def k(y_hbm, z_hbm, dst_hbm, g_hbm, w_hbm, s_hbm, t_hbm, out_hbm,
          acc, dstv, gv, wv, msg, dv, svv, tvv, sem):
        h = lax.axis_index("c")
        tec = lax.axis_index("s")
        rowb = tec * rows_t
        pltpu.sync_copy(z_hbm.at[h, pl.ds(rowb, rows_t)],
                        acc.at[pl.ds(rowb, rows_t)])
        pltpu.sync_copy(s_hbm.at[h], svv)
        pltpu.sync_copy(t_hbm.at[h], tvv)
        plsc.subcore_barrier()

        ebase = tec * (EPAD // NS)
        yoff = h * 16

        def chunk(ch, _):
            cb = ebase + ch * eca
            pltpu.sync_copy(dst_hbm.at[pl.ds(cb, eca)], dstv)
            pltpu.sync_copy(g_hbm.at[pl.ds(cb, eca)], gv)
            pltpu.sync_copy(w_hbm.at[pl.ds(cb, eca)], wv)

            def addoff(i, _):
                sl = pl.ds(i * 16, 16)
                gv[sl] = gv[sl] + yoff
                return 0
            lax.fori_loop(0, eca // 16, addoff, 0)
            pltpu.async_copy(y_hbm.at[gv], msg, sem).wait()

            def scale16(i, _):
                w16 = wv[pl.ds(i * 16, 16)]
                base = i * 16
                for j in range(16):
                    wsc = w16[j]
                    msg[base + j, 0:16] = msg[base + j, 0:16] * wsc
                    msg[base + j, 16:32] = msg[base + j, 16:32] * wsc
                return 0
            lax.fori_loop(0, eca // 16, scale16, 0)
            pltpu.sync_copy(msg, acc.at[dstv], add=True)
            return 0
        lax.fori_loop(0, EPAD // NS // eca, chunk, 0)
        plsc.subcore_barrier()

        slo, shi = svv[0:16], svv[16:32]
        tlo, thi = tvv[0:16], tvv[16:32]

        def drain(d, _):
            rb = rowb + d * drows
            pltpu.sync_copy(acc.at[pl.ds(rb, drows)], dv)

            def bnrow(i, _):
                dv[i, 0:16] = dv[i, 0:16] * slo + tlo
                dv[i, 16:32] = dv[i, 16:32] * shi + thi
                return 0
            lax.fori_loop(0, drows, bnrow, 0)
            pltpu.sync_copy(dv, out_hbm.at[pl.ds(rb, drows),
                                           pl.ds(h * 32, 32)])
            return 0
        lax.fori_loop(0, rows_t // drows, drain, 0)

    return k(Yflat, Z, dstp, gp, wp, sfin, tfin)


def _pack_weights(W, root):
    halves = []
    for h in range(2):
        halves.append(jnp.concatenate(
            [W[r][:, h * 32:(h + 1) * 32] for r in range(R)], axis=1))
    return jnp.concatenate(halves + [root], axis=1)


def kernel(n_id, edge_index, edge_types, emb_table,
           bn32_g, bn32_b, W1, root1, b1, bn64_g, bn64_b,
           W2, root2, b2, bn64_2_g, bn64_2_b):
    inv = 1.0 / jnp.sqrt(1.0 + EPS)
    idx = jnp.pad(n_id.astype(jnp.int32), (0, NPAD - N))
    srcp = jnp.pad(edge_index[0].astype(jnp.int32), (0, EPAD - E))
    dstp = jnp.pad(edge_index[1].astype(jnp.int32), (0, EPAD - E),
                   constant_values=N)
    etp = jnp.pad(edge_types[:E].astype(jnp.int32), (0, EPAD - E))

    x0 = _emb_gather(emb_table, idx)
    cnt = _seg_counts(dstp, etp)
    w, g = _edge_weights(srcp, dstp, etp, cnt)

    ones2 = jnp.ones((2, 32), jnp.float32)
    zeros2 = jnp.zeros((2, 32), jnp.float32)
    sfin = (bn64_2_g * inv).reshape(2, 32)
    tfin = bn64_2_b.reshape(2, 32)

    Y1, Z1 = _tc_transform(x0, _pack_weights(W1, root1), b1,
                           bn32_g * inv, bn32_b, relu_pre=False)
    h1 = _sc_aggregate(Y1.reshape(NPAD * 32, 32), Z1,
                       dstp, g, w, ones2, zeros2)

    Y2, Z2 = _tc_transform(h1, _pack_weights(W2, root2), b2,
                           bn64_g * inv, bn64_b, relu_pre=True)
    h2 = _sc_aggregate(Y2.reshape(NPAD * 32, 32), Z2,
                       dstp, g, w, sfin, tfin)
    return h2[:N]


# speedup vs baseline: 5.7182x; 1.0800x over previous
import functools

import jax
import jax.numpy as jnp
from jax import lax
from jax.experimental import pallas as pl
from jax.experimental.pallas import tpu as pltpu
from jax.experimental.pallas import tpu_sc as plsc

NC, NS, L = 2, 16, 16
NW = NC * NS

N = 50000
E = 800000
R = 16
D_IN = 32
D_HID = 64
EPS = 1e-5

NPAD = 50176
EPAD = 800768
EPAD2 = 851968
EC = 1472
YROWS = 36

HLOC = 400128
HREAL = N * R // 2
HDUMMY = 400064
BN = 512

_mesh = plsc.VectorSubcoreMesh(
    core_axis_name="c", subcore_axis_name="s", num_cores=NC, num_subcores=NS)
_sc_params = pltpu.CompilerParams(use_tc_tiling_on_sc=False)


def _emb_gather(table, idx):
    b_per_w = NPAD // NW

    @functools.partial(
        pl.kernel, mesh=_mesh,
        out_type=jax.ShapeDtypeStruct((NPAD, D_IN), jnp.float32),
        compiler_params=_sc_params,
        scratch_types=[
            pltpu.VMEM((b_per_w,), jnp.int32),
            pltpu.VMEM((b_per_w, D_IN), jnp.float32),
            pltpu.SemaphoreType.DMA,
        ],
    )
    def k(table_hbm, idx_hbm, out_hbm, idx_v, rows_v, sem):
        wid = lax.axis_index("s") * NC + lax.axis_index("c")
        base = wid * b_per_w
        pltpu.sync_copy(idx_hbm.at[pl.ds(base, b_per_w)], idx_v)
        pltpu.async_copy(table_hbm.at[idx_v], rows_v, sem).wait()
        pltpu.sync_copy(rows_v, out_hbm.at[pl.ds(base, b_per_w)])

    return k(table, idx)


def _seg_counts(dstp, etp):
    @functools.partial(
        pl.kernel, mesh=_mesh,
        out_type=jax.ShapeDtypeStruct((N * R + 8,), jnp.float32),
        compiler_params=_sc_params,
        scratch_types=[
            pltpu.VMEM_SHARED((HLOC,), jnp.float32),
            pltpu.VMEM((HLOC // NS,), jnp.float32),
            pltpu.VMEM((EC,), jnp.int32),
            pltpu.VMEM((EC,), jnp.int32),
            pltpu.VMEM((EC,), jnp.int32),
            pltpu.VMEM((EC,), jnp.float32),
        ],
    )
    def k(dst_hbm, et_hbm, cnt_hbm, hist, zbuf, dstv, etv, idxv, onesv):
        c = lax.axis_index("c")
        tec = lax.axis_index("s")
        zslice = HLOC // NS

        def z16(i, _):
            zbuf[pl.ds(i * 16, 16)] = jnp.zeros((16,), jnp.float32)
            return 0
        lax.fori_loop(0, zslice // 16, z16, 0)
        pltpu.sync_copy(zbuf, hist.at[pl.ds(tec * zslice, zslice)])

        def o16(i, _):
            onesv[pl.ds(i * 16, 16)] = jnp.ones((16,), jnp.float32)
            return 0
        lax.fori_loop(0, EC // 16, o16, 0)
        plsc.subcore_barrier()

        lo = c * HREAL
        ebase = tec * (EPAD // NS)

        def chunk(ch, _):
            cb = ebase + ch * EC
            pltpu.sync_copy(dst_hbm.at[pl.ds(cb, EC)], dstv)
            pltpu.sync_copy(et_hbm.at[pl.ds(cb, EC)], etv)

            def cmp16(i, _):
                sl = pl.ds(i * 16, 16)
                seg = dstv[sl] * R + etv[sl] - lo
                ok = (seg >= 0) & (seg < HREAL)
                idxv[sl] = jnp.where(ok, seg, HDUMMY)
                return 0
            lax.fori_loop(0, EC // 16, cmp16, 0)
            pltpu.sync_copy(onesv, hist.at[idxv], add=True)
            return 0
        lax.fori_loop(0, EPAD // NS // EC, chunk, 0)
        plsc.subcore_barrier()

        wslice = HREAL // NS
        pltpu.sync_copy(hist.at[pl.ds(tec * wslice, wslice)],
                        cnt_hbm.at[pl.ds(c * HREAL + tec * wslice, wslice)])

    return k(dstp, etp)


def _edge_weights(srcp, dstp, etp, cnt):
    e_per_w = EPAD // NW

    @functools.partial(
        pl.kernel, mesh=_mesh,
        out_type=[jax.ShapeDtypeStruct((EPAD,), jnp.float32),
                  jax.ShapeDtypeStruct((EPAD,), jnp.int32)],
        compiler_params=_sc_params,
        scratch_types=[
            pltpu.VMEM((EC,), jnp.int32),
            pltpu.VMEM((EC,), jnp.int32),
            pltpu.VMEM((EC,), jnp.int32),
            pltpu.VMEM((EC,), jnp.int32),
            pltpu.VMEM((EC,), jnp.float32),
            pltpu.VMEM((EC,), jnp.float32),
            pltpu.VMEM((EC,), jnp.int32),
            pltpu.SemaphoreType.DMA,
        ],
    )
    def k(src_hbm, dst_hbm, et_hbm, cnt_hbm, w_hbm, g_hbm,
          srcv, dstv, etv, segv, cv, wv, gvb, sem):
        wid = lax.axis_index("s") * NC + lax.axis_index("c")
        wbase = wid * e_per_w

        def chunk(ch, _):
            cb = wbase + ch * EC
            pltpu.sync_copy(src_hbm.at[pl.ds(cb, EC)], srcv)
            pltpu.sync_copy(dst_hbm.at[pl.ds(cb, EC)], dstv)
            pltpu.sync_copy(et_hbm.at[pl.ds(cb, EC)], etv)

            def cmp16(i, _):
                sl = pl.ds(i * 16, 16)
                segv[sl] = dstv[sl] * R + etv[sl]
                gvb[sl] = srcv[sl] * YROWS + etv[sl]
                return 0
            lax.fori_loop(0, EC // 16, cmp16, 0)
            pltpu.async_copy(cnt_hbm.at[segv], cv, sem).wait()

            def inv16(i, _):
                sl = pl.ds(i * 16, 16)
                wv[sl] = 1.0 / cv[sl]
                return 0
            lax.fori_loop(0, EC // 16, inv16, 0)
            pltpu.sync_copy(wv, w_hbm.at[pl.ds(cb, EC)])
            pltpu.sync_copy(gvb, g_hbm.at[pl.ds(cb, EC)])
            return 0
        lax.fori_loop(0, e_per_w // EC, chunk, 0)

    return k(srcp, dstp, etp, cnt)


def _tc_transform(x, Wcat, spre, tpre, relu_pre):
    d_in = x.shape[1]

    def body(x_ref, w_ref, s_ref, t_ref, y_ref):
        xb = x_ref[...]
        if relu_pre:
            xb = jnp.maximum(xb, 0.0)
        xb = xb * s_ref[...] + t_ref[...]
        ycat = jnp.dot(xb, w_ref[...], preferred_element_type=jnp.float32)
        y_ref[...] = ycat.reshape(BN * 9, 128)

    return pl.pallas_call(
        body,
        grid=(NPAD // BN,),
        in_specs=[pl.BlockSpec((BN, d_in), lambda i: (i, 0)),
                  pl.BlockSpec((d_in, 1152), lambda i: (0, 0)),
                  pl.BlockSpec((1, d_in), lambda i: (0, 0)),
                  pl.BlockSpec((1, d_in), lambda i: (0, 0))],
        out_specs=pl.BlockSpec((BN * 9, 128), lambda i: (i, 0)),
        out_shape=jax.ShapeDtypeStruct((NPAD * 9, 128), jnp.float32),
    )(x, Wcat, spre.reshape(1, d_in), tpre.reshape(1, d_in))


def _sc_aggregate(Yflat, dstp, gp, wp, sfin, tfin):
    rows_t = NPAD // NS
    drows = 224
    eca = 512

    @functools.partial(
        pl.kernel, mesh=_mesh,
        out_type=jax.ShapeDtypeStruct((NPAD, D_HID), jnp.float32),
        compiler_params=_sc_params,
        scratch_types=[
            pltpu.VMEM_SHARED((NPAD, 32), jnp.float32),
            pltpu.VMEM((eca,), jnp.int32),
            pltpu.VMEM((eca,), jnp.int32),
            pltpu.VMEM((eca,), jnp.float32),
            pltpu.VMEM((eca, 32), jnp.float32),
            pltpu.VMEM((drows, 32), jnp.float32),
            pltpu.VMEM((32,), jnp.float32),
            pltpu.VMEM((32,), jnp.float32),
            pltpu.SemaphoreType.DMA,
        ],
    )
    def k(y_hbm, dst_hbm, g_hbm, w_hbm, s_hbm, t_hbm, out_hbm,
          acc, dstv, gv, wv, msg, dv, svv, tvv, sem):
        h = lax.axis_index("c")
        tec = lax.axis_index("s")
        rowb = tec * rows_t

        def zrow(i, _):
            dv[i, 0:16] = jnp.zeros((16,), jnp.float32)
            dv[i, 16:32] = jnp.zeros((16,), jnp.float32)
            return 0
        lax.fori_loop(0, drows, zrow, 0)

        def zcp(d, _):
            pltpu.sync_copy(dv, acc.at[pl.ds(rowb + d * drows, drows)])
            return 0
        lax.fori_loop(0, rows_t // drows, zcp, 0)
        pltpu.sync_copy(s_hbm.at[h], svv)
        pltpu.sync_copy(t_hbm.at[h], tvv)
        plsc.subcore_barrier()

        ebase = tec * (EPAD2 // NS)
        yoff = h * 17

        def chunk(ch, _):
            cb = ebase + ch * eca
            pltpu.sync_copy(dst_hbm.at[pl.ds(cb, eca)], dstv)
            pltpu.sync_copy(g_hbm.at[pl.ds(cb, eca)], gv)
            pltpu.sync_copy(w_hbm.at[pl.ds(cb, eca)], wv)

            def addoff(i, _):
                sl = pl.ds(i * 16, 16)
                gv[sl] = gv[sl] + yoff
                return 0
            lax.fori_loop(0, eca // 16, addoff, 0)
            pltpu.async_copy(y_hbm.at[gv], msg, sem).wait()

            def scale16(i, _):
                w16 = wv[pl.ds(i * 16, 16)]
                base = i * 16
                for j in range(16):
                    wsc = w16[j]
                    msg[base + j, 0:16] = msg[base + j, 0:16] * wsc
                    msg[base + j, 16:32] = msg[base + j, 16:32] * wsc
                return 0
            lax.fori_loop(0, eca // 16, scale16, 0)
            pltpu.sync_copy(msg, acc.at[dstv], add=True)
            return 0
        lax.fori_loop(0, EPAD2 // NS // eca, chunk, 0)
        plsc.subcore_barrier()

        slo, shi = svv[0:16], svv[16:32]
        tlo, thi = tvv[0:16], tvv[16:32]

        def drain(d, _):
            rb = rowb + d * drows
            pltpu.sync_copy(acc.at[pl.ds(rb, drows)], dv)

            def bnrow(i, _):
                dv[i, 0:16] = dv[i, 0:16] * slo + tlo
                dv[i, 16:32] = dv[i, 16:32] * shi + thi
                return 0
            lax.fori_loop(0, drows, bnrow, 0)
            pltpu.sync_copy(dv, out_hbm.at[pl.ds(rb, drows),
                                           pl.ds(h * 32, 32)])
            return 0
        lax.fori_loop(0, rows_t // drows, drain, 0)

    return k(Yflat, dstp, gp, wp, sfin, tfin)


def _pack_weights(W, root):
    halves = []
    for h in range(2):
        halves.append(jnp.concatenate(
            [W[r][:, h * 32:(h + 1) * 32] for r in range(R)]
            + [root[:, h * 32:(h + 1) * 32]], axis=1))
    d_in = root.shape[0]
    return jnp.concatenate(
        halves + [jnp.zeros((d_in, 64), jnp.float32)], axis=1)


def kernel(n_id, edge_index, edge_types, emb_table,
           bn32_g, bn32_b, W1, root1, b1, bn64_g, bn64_b,
           W2, root2, b2, bn64_2_g, bn64_2_b):
    inv = 1.0 / jnp.sqrt(1.0 + EPS)
    idx = jnp.pad(n_id.astype(jnp.int32), (0, NPAD - N))
    srcp = jnp.pad(edge_index[0].astype(jnp.int32), (0, EPAD - E))
    dstp = jnp.pad(edge_index[1].astype(jnp.int32), (0, EPAD - E),
                   constant_values=N)
    etp = jnp.pad(edge_types[:E].astype(jnp.int32), (0, EPAD - E))

    x0 = _emb_gather(emb_table, idx)
    cnt = _seg_counts(dstp, etp)
    w, g = _edge_weights(srcp, dstp, etp, cnt)

    padv = EPAD2 - E - N
    ar = jnp.arange(N, dtype=jnp.int32)
    dst2 = jnp.concatenate([edge_index[1].astype(jnp.int32), ar,
                            jnp.full((padv,), N, jnp.int32)])
    g2 = jnp.concatenate([g[:E], ar * YROWS + 16,
                          jnp.zeros((padv,), jnp.int32)])
    w2 = jnp.concatenate([w[:E], jnp.ones((N,), jnp.float32),
                          jnp.zeros((padv,), jnp.float32)])

    ones2 = jnp.ones((2, 32), jnp.float32)
    sfin = (bn64_2_g * inv).reshape(2, 32)
    tfin = (bn64_2_b + bn64_2_g * inv * b2).reshape(2, 32)

    Y1 = _tc_transform(x0, _pack_weights(W1, root1),
                       bn32_g * inv, bn32_b, relu_pre=False)
    h1 = _sc_aggregate(Y1.reshape(NPAD * YROWS, 32),
                       dst2, g2, w2, ones2, b1.reshape(2, 32))

    Y2 = _tc_transform(h1, _pack_weights(W2, root2),
                       bn64_g * inv, bn64_b, relu_pre=True)
    h2 = _sc_aggregate(Y2.reshape(NPAD * YROWS, 32),
                       dst2, g2, w2, sfin, tfin)
    return h2[:N]
